# Initial kernel scaffold; baseline (speedup 1.0000x reference)
import jax
import jax.numpy as jnp
from jax.experimental import pallas as pl


def kernel(v_size, c_size, v_edge_index, c_edge_index, p_edge_index, n_edge_index, v_emb, c_emb, params):
    raise NotImplementedError("write your pallas kernel here")


# speedup vs baseline: 3.5100x; 3.5100x over previous
import functools

import jax
import jax.numpy as jnp
from jax import lax
from jax.experimental import pallas as pl
from jax.experimental.pallas import tpu as pltpu
from jax.experimental.pallas import tpu_sc as plsc

f32 = jnp.float32
i32 = jnp.int32

NC, NS = 2, 16
NW = NC * NS
CHUNK = 128
DIM = 128
BR = 2560


def _mesh():
    return plsc.VectorSubcoreMesh(core_axis_name="c", subcore_axis_name="s")


def _sc_precompute(ve_pad, ce_pad, p_idx, n_idx, ones2d, zeros2d, R):
    NCH = p_idx.shape[1]
    rows = R // NS
    out_type = (
        jax.ShapeDtypeStruct((NW, NCH, CHUNK), i32),
        jax.ShapeDtypeStruct((NW, NCH, CHUNK), i32),
        jax.ShapeDtypeStruct((NW, NCH, CHUNK), i32),
        jax.ShapeDtypeStruct((NW, NCH, CHUNK), i32),
        jax.ShapeDtypeStruct((4, NC, R, CHUNK), f32),
    )
    scratch = [
        pltpu.VMEM((NCH, CHUNK), i32),
        pltpu.VMEM((NCH, CHUNK), i32),
        pltpu.VMEM((NCH, CHUNK), i32),
        pltpu.VMEM((CHUNK, CHUNK), f32),
        pltpu.VMEM((16, CHUNK), f32),
        pltpu.VMEM_SHARED((R, CHUNK), f32),
    ]

    @functools.partial(pl.kernel, out_type=out_type, mesh=_mesh(),
                       scratch_types=scratch)
    def k(ve, ce, pidx, nidx, ones_h, zeros_h,
          vp_o, cp_o, vn_o, cn_o, deg_o,
          eidx_v, iv_v, ic_v, ones_v, z_v, acc):
        cid = lax.axis_index("c")
        sid = lax.axis_index("s")
        w = sid * NC + cid
        base = sid * rows
        pltpu.sync_copy(ones_h, ones_v)
        pltpu.sync_copy(zeros_h, z_v)
        for kk in range(rows // 16):
            pltpu.sync_copy(z_v, acc.at[pl.ds(base + kk * 16, 16), :])
        plsc.subcore_barrier()

        for side, (idx_h, v_o, c_o) in enumerate(
                ((pidx, vp_o, cp_o), (nidx, vn_o, cn_o))):
            pltpu.sync_copy(idx_h.at[w], eidx_v)

            def gbody(j, carry):
                ej = eidx_v.at[j]
                pltpu.sync_copy(ve.at[ej], iv_v.at[j])
                pltpu.sync_copy(ce.at[ej], ic_v.at[j])
                return carry

            lax.fori_loop(0, NCH, gbody, 0)
            pltpu.sync_copy(iv_v, v_o.at[w])
            pltpu.sync_copy(ic_v, c_o.at[w])

            for half, idxref in ((0, iv_v), (1, ic_v)):
                a = 2 * side + half

                def hbody(j, carry, idxref=idxref):
                    pltpu.sync_copy(ones_v, acc.at[idxref.at[j]], add=True)
                    return carry

                lax.fori_loop(0, NCH, hbody, 0)
                plsc.subcore_barrier()
                for kk in range(rows // CHUNK):
                    sl = pl.ds(base + kk * CHUNK, CHUNK)
                    pltpu.sync_copy(acc.at[sl, :], deg_o.at[a, cid, sl, :])
                for kk in range(rows // 16):
                    pltpu.sync_copy(z_v, acc.at[pl.ds(base + kk * 16, 16), :])
                plsc.subcore_barrier()

    return k(ve_pad, ce_pad, p_idx, n_idx, ones2d, zeros2d)


def _sc_edge(Mp, Mn, Mpc, Mnc, vp, cp, vn, cn, zeros2d, R):
    NCH = vp.shape[1]
    rows = R // NS
    out_type = tuple(jax.ShapeDtypeStruct((NC, R, DIM), f32) for _ in range(4))
    scratch = [
        pltpu.VMEM((NCH, CHUNK), i32),
        pltpu.VMEM((NCH, CHUNK), i32),
        pltpu.VMEM((NCH, CHUNK), i32),
        pltpu.VMEM((NCH, CHUNK), i32),
        pltpu.VMEM((CHUNK, DIM), f32),
        pltpu.VMEM((16, DIM), f32),
        pltpu.VMEM_SHARED((R, DIM), f32),
    ]

    @functools.partial(pl.kernel, out_type=out_type, mesh=_mesh(),
                       scratch_types=scratch)
    def k(mp, mn, mpc, mnc, vp_h, cp_h, vn_h, cn_h, z_h,
          pA, nA, pC, nC,
          ivp, icp, ivn, icn, rows_v, z_v, acc):
        cid = lax.axis_index("c")
        sid = lax.axis_index("s")
        w = sid * NC + cid
        base = sid * rows
        pltpu.sync_copy(vp_h.at[w], ivp)
        pltpu.sync_copy(cp_h.at[w], icp)
        pltpu.sync_copy(vn_h.at[w], ivn)
        pltpu.sync_copy(cn_h.at[w], icn)
        pltpu.sync_copy(z_h, z_v)
        for kk in range(rows // 16):
            pltpu.sync_copy(z_v, acc.at[pl.ds(base + kk * 16, 16), :])
        plsc.subcore_barrier()
        for tab, isrc, idst, out in ((mp, ivp, icp, pA), (mn, ivn, icn, nA),
                                     (mpc, icp, ivp, pC), (mnc, icn, ivn, nC)):
            def body(j, carry, tab=tab, isrc=isrc, idst=idst):
                pltpu.sync_copy(tab.at[isrc.at[j]], rows_v)
                pltpu.sync_copy(rows_v, acc.at[idst.at[j]], add=True)
                return carry
            lax.fori_loop(0, NCH, body, 0)
            plsc.subcore_barrier()
            for kk in range(rows // CHUNK):
                sl = pl.ds(base + kk * CHUNK, CHUNK)
                pltpu.sync_copy(acc.at[sl, :], rows_v)
                pltpu.sync_copy(rows_v, out.at[cid].at[sl, :])
            for kk in range(rows // 16):
                pltpu.sync_copy(z_v, acc.at[pl.ds(base + kk * 16, 16), :])
            plsc.subcore_barrier()

    return k(Mp, Mn, Mpc, Mnc, vp, cp, vn, cn, zeros2d)


def _tc_scales(degp, zoffs, R):

    def body(d, z, o):
        dv = d[...]
        cnt = dv[:, 0, :, 0:1] + dv[:, 1, :, 0:1] + z[...]
        o[...] = lax.rsqrt(jnp.maximum(cnt, 1.0))

    return pl.pallas_call(
        body,
        grid=(R // BR,),
        in_specs=[pl.BlockSpec((4, NC, BR, CHUNK), lambda i: (0, 0, i, 0)),
                  pl.BlockSpec((4, 1, 1), lambda i: (0, 0, 0))],
        out_specs=pl.BlockSpec((4, BR, 1), lambda i: (0, i, 0)),
        out_shape=jax.ShapeDtypeStruct((4, R, 1), f32),
    )(degp, zoffs)


def _tc_msg(X, pa, pb, sa, sb):
    R = X.shape[0]

    def body(x, w1a, b1a, w2a, b2a, w1b, b1b, w2b, b2b, sa_r, sb_r, oa, ob):
        xv = x[...]
        ha = jnp.maximum(jnp.dot(xv, w1a[...], preferred_element_type=f32)
                         + b1a[...], 0.0)
        oa[...] = (jnp.dot(ha, w2a[...], preferred_element_type=f32)
                   + b2a[...]) * sa_r[...]
        hb = jnp.maximum(jnp.dot(xv, w1b[...], preferred_element_type=f32)
                         + b1b[...], 0.0)
        ob[...] = (jnp.dot(hb, w2b[...], preferred_element_type=f32)
                   + b2b[...]) * sb_r[...]

    row = lambda i: (i, 0)
    full = lambda i: (0, 0)
    wspec = pl.BlockSpec((DIM, DIM), full)
    bspec = pl.BlockSpec((1, DIM), full)
    dspec = pl.BlockSpec((BR, 1), row)
    return pl.pallas_call(
        body,
        grid=(R // BR,),
        in_specs=[pl.BlockSpec((BR, DIM), row),
                  wspec, bspec, wspec, bspec,
                  wspec, bspec, wspec, bspec,
                  dspec, dspec],
        out_specs=[pl.BlockSpec((BR, DIM), row)] * 2,
        out_shape=[jax.ShapeDtypeStruct((R, DIM), f32)] * 2,
    )(X, pa["W1"], pa["b1"].reshape(1, DIM), pa["W2"], pa["b2"].reshape(1, DIM),
      pb["W1"], pb["b1"].reshape(1, DIM), pb["W2"], pb["b2"].reshape(1, DIM),
      sa, sb)


def _tc_update(X, A, B, W, bvec, sa, sb):
    R = X.shape[0]

    def body(x, a, b, w0, w1, w2, bi, sa_r, sb_r, o):
        am = (a[0] + a[1]) * sa_r[...]
        bm = (b[0] + b[1]) * sb_r[...]
        o[...] = (jnp.dot(x[...], w0[...], preferred_element_type=f32)
                  + jnp.dot(am, w1[...], preferred_element_type=f32)
                  + jnp.dot(bm, w2[...], preferred_element_type=f32)
                  + bi[...])

    row = lambda i: (i, 0)
    full = lambda i: (0, 0)
    prow = lambda i: (0, i, 0)
    wspec = pl.BlockSpec((DIM, DIM), full)
    dspec = pl.BlockSpec((BR, 1), row)
    return pl.pallas_call(
        body,
        grid=(R // BR,),
        in_specs=[pl.BlockSpec((BR, DIM), row),
                  pl.BlockSpec((NC, BR, DIM), prow),
                  pl.BlockSpec((NC, BR, DIM), prow),
                  wspec, wspec, wspec,
                  pl.BlockSpec((1, DIM), full),
                  dspec, dspec],
        out_specs=pl.BlockSpec((BR, DIM), row),
        out_shape=jax.ShapeDtypeStruct((R, DIM), f32),
    )(X, A, B, W[0], W[1], W[2], bvec.reshape(1, DIM), sa, sb)


def kernel(v_size, c_size, v_edge_index, c_edge_index, p_edge_index,
           n_edge_index, v_emb, c_emb, params):
    V = v_emb.shape[0]
    Cn = c_emb.shape[0]
    E = v_edge_index.shape[0]
    EP = p_edge_index.shape[0]
    TRASH = max(V, Cn)
    R = -(-(TRASH + 1) // BR) * BR

    EPW = EP // NW
    NCH = -(-EPW // CHUNK)
    PADW = NCH * CHUNK - EPW

    ve_pad = jnp.concatenate(
        [v_edge_index.astype(i32), jnp.full((64,), TRASH, i32)])
    ce_pad = jnp.concatenate(
        [c_edge_index.astype(i32), jnp.full((64,), TRASH, i32)])
    p_idx = jnp.concatenate(
        [p_edge_index.astype(i32).reshape(NW, EPW),
         jnp.full((NW, PADW), E, i32)], axis=1).reshape(NW, NCH, CHUNK)
    n_idx = jnp.concatenate(
        [n_edge_index.astype(i32).reshape(NW, EPW),
         jnp.full((NW, PADW), E, i32)], axis=1).reshape(NW, NCH, CHUNK)

    zeros2d = jnp.zeros((16, DIM), f32)
    ones2d = jnp.ones((CHUNK, CHUNK), f32)

    vp, cp, vn, cn, degp = _sc_precompute(ve_pad, ce_pad, p_idx, n_idx,
                                          ones2d, zeros2d, R)

    v_zero = jnp.asarray(v_size - V, f32).reshape(1, 1)
    c_zero = jnp.asarray(c_size - Cn, f32).reshape(1, 1)
    zoffs = jnp.stack([v_zero, c_zero, v_zero, c_zero]).reshape(4, 1, 1)
    scales = _tc_scales(degp, zoffs, R)
    spv, spc, snv, snc = scales[0], scales[1], scales[2], scales[3]

    vW = params["vU_W"].reshape(3, DIM, DIM)
    cW = params["cU_W"].reshape(3, DIM, DIM)

    v_cur = jnp.concatenate([v_emb, jnp.zeros((R - V, DIM), f32)])
    c_cur = jnp.concatenate([c_emb, jnp.zeros((R - Cn, DIM), f32)])
    v_list = [v_cur]
    c_list = [c_cur]
    for _ in range(4):
        Mp, Mn = _tc_msg(v_cur, params["p_v2c"], params["n_v2c"], spv, snv)
        Mpc, Mnc = _tc_msg(c_cur, params["p_c2v"], params["n_c2v"], spc, snc)
        pA, nA, pC, nC = _sc_edge(Mp, Mn, Mpc, Mnc, vp, cp, vn, cn,
                                  zeros2d, R)
        c_cur = _tc_update(c_cur, pA, nA, cW, params["cU_b"], spc, snc)
        v_cur = _tc_update(v_cur, pC, nC, vW, params["vU_b"], spv, snv)
        c_list.append(c_cur)
        v_list.append(v_cur)

    v_out = jnp.stack(v_list)[:, :V, :]
    c_out = jnp.stack(c_list)[:, :Cn, :]
    return (v_out, c_out)


# speedup vs baseline: 3.8340x; 1.0923x over previous
import functools

import jax
import jax.numpy as jnp
from jax import lax
from jax.experimental import pallas as pl
from jax.experimental.pallas import tpu as pltpu
from jax.experimental.pallas import tpu_sc as plsc

f32 = jnp.float32
i32 = jnp.int32

NC, NS = 2, 16
NW = NC * NS
CHUNK = 128
DIM = 128
BR = 2560


def _mesh():
    return plsc.VectorSubcoreMesh(core_axis_name="c", subcore_axis_name="s")


def _sc_precompute(ve_pad, ce_pad, p_idx, n_idx, ones2d, zeros2d, R):
    NCH = p_idx.shape[1]
    rows = R // NS
    out_type = (
        jax.ShapeDtypeStruct((NW, NCH, CHUNK), i32),
        jax.ShapeDtypeStruct((NW, NCH, CHUNK), i32),
        jax.ShapeDtypeStruct((NW, NCH, CHUNK), i32),
        jax.ShapeDtypeStruct((NW, NCH, CHUNK), i32),
        jax.ShapeDtypeStruct((4, NC, R, CHUNK), f32),
    )
    scratch = [
        pltpu.VMEM((NCH, CHUNK), i32),
        pltpu.VMEM((NCH, CHUNK), i32),
        pltpu.VMEM((NCH, CHUNK), i32),
        pltpu.VMEM((CHUNK, CHUNK), f32),
        pltpu.VMEM((32, CHUNK), f32),
        pltpu.VMEM_SHARED((R, CHUNK), f32),
    ]

    @functools.partial(pl.kernel, out_type=out_type, mesh=_mesh(),
                       scratch_types=scratch)
    def k(ve, ce, pidx, nidx, ones_h, zeros_h,
          vp_o, cp_o, vn_o, cn_o, deg_o,
          eidx_v, iv_v, ic_v, ones_v, z_v, acc):
        cid = lax.axis_index("c")
        sid = lax.axis_index("s")
        w = sid * NC + cid
        base = sid * rows
        pltpu.sync_copy(ones_h, ones_v)
        pltpu.sync_copy(zeros_h, z_v)
        for kk in range(rows // 32):
            pltpu.sync_copy(z_v, acc.at[pl.ds(base + kk * 32, 32), :])
        plsc.subcore_barrier()

        for side, (idx_h, v_o, c_o) in enumerate(
                ((pidx, vp_o, cp_o), (nidx, vn_o, cn_o))):
            pltpu.sync_copy(idx_h.at[w], eidx_v)

            def gbody(j, carry):
                ej = eidx_v.at[j]
                pltpu.sync_copy(ve.at[ej], iv_v.at[j])
                pltpu.sync_copy(ce.at[ej], ic_v.at[j])
                return carry

            lax.fori_loop(0, NCH, gbody, 0)
            pltpu.sync_copy(iv_v, v_o.at[w])
            pltpu.sync_copy(ic_v, c_o.at[w])

            for half, idxref in ((0, iv_v), (1, ic_v)):
                a = 2 * side + half

                def hbody(j, carry, idxref=idxref):
                    pltpu.sync_copy(ones_v, acc.at[idxref.at[j]], add=True)
                    return carry

                lax.fori_loop(0, NCH, hbody, 0)
                plsc.subcore_barrier()
                for kk in range(rows // CHUNK):
                    sl = pl.ds(base + kk * CHUNK, CHUNK)
                    pltpu.sync_copy(acc.at[sl, :], deg_o.at[a, cid, sl, :])
                for kk in range(rows // 32):
                    pltpu.sync_copy(z_v, acc.at[pl.ds(base + kk * 32, 32), :])
                plsc.subcore_barrier()

    return k(ve_pad, ce_pad, p_idx, n_idx, ones2d, zeros2d)


def _sc_edge(Mp, Mn, Mpc, Mnc, vp, cp, vn, cn, zeros2d, R):
    NCH = vp.shape[1]
    rows = R // NS
    out_type = tuple(jax.ShapeDtypeStruct((NC, R, DIM), f32) for _ in range(4))
    scratch = [
        pltpu.VMEM((NCH, CHUNK), i32),
        pltpu.VMEM((NCH, CHUNK), i32),
        pltpu.VMEM((2, CHUNK, DIM), f32),
        pltpu.VMEM((32, DIM), f32),
        pltpu.SemaphoreType.DMA,
        pltpu.VMEM_SHARED((R, DIM), f32),
    ]

    @functools.partial(pl.kernel, out_type=out_type, mesh=_mesh(),
                       scratch_types=scratch)
    def k(mp, mn, mpc, mnc, vp_h, cp_h, vn_h, cn_h, z_h,
          pA, nA, pC, nC,
          isrc, idst, rv, z_v, gsem, acc):
        cid = lax.axis_index("c")
        sid = lax.axis_index("s")
        w = sid * NC + cid
        base = sid * rows
        pltpu.sync_copy(z_h, z_v)
        for kk in range(rows // 32):
            pltpu.sync_copy(z_v, acc.at[pl.ds(base + kk * 32, 32), :])
        plsc.subcore_barrier()
        for tab, s_h, d_h, out in ((mp, vp_h, cp_h, pA), (mpc, cp_h, vp_h, pC),
                                   (mn, vn_h, cn_h, nA), (mnc, cn_h, vn_h, nC)):
            pltpu.sync_copy(s_h.at[w], isrc)
            pltpu.sync_copy(d_h.at[w], idst)
            pltpu.async_copy(tab.at[isrc.at[0]], rv.at[0], gsem)

            def body(j, carry, tab=tab):
                ph = lax.bitwise_and(j, 1)
                pltpu.make_async_copy(tab.at[isrc.at[j]], rv.at[ph],
                                      gsem).wait()

                @pl.when(j + 1 < NCH)
                def _():
                    pltpu.async_copy(tab.at[isrc.at[j + 1]], rv.at[1 - ph],
                                     gsem)

                pltpu.sync_copy(rv.at[ph], acc.at[idst.at[j]], add=True)
                return carry

            lax.fori_loop(0, NCH, body, 0)
            plsc.subcore_barrier()
            for kk in range(rows // CHUNK):
                sl = pl.ds(base + kk * CHUNK, CHUNK)
                pltpu.sync_copy(acc.at[sl, :], rv.at[0])
                pltpu.sync_copy(rv.at[0], out.at[cid].at[sl, :])
            for kk in range(rows // 32):
                pltpu.sync_copy(z_v, acc.at[pl.ds(base + kk * 32, 32), :])
            plsc.subcore_barrier()

    return k(Mp, Mn, Mpc, Mnc, vp, cp, vn, cn, zeros2d)


def _tc_scales(degp, zoffs, R):

    def body(d, z, o):
        dv = d[...]
        cnt = dv[:, 0, :, 0:1] + dv[:, 1, :, 0:1] + z[...]
        o[...] = lax.rsqrt(jnp.maximum(cnt, 1.0))

    return pl.pallas_call(
        body,
        grid=(R // BR,),
        in_specs=[pl.BlockSpec((4, NC, BR, CHUNK), lambda i: (0, 0, i, 0)),
                  pl.BlockSpec((4, 1, 1), lambda i: (0, 0, 0))],
        out_specs=pl.BlockSpec((4, BR, 1), lambda i: (0, i, 0)),
        out_shape=jax.ShapeDtypeStruct((4, R, 1), f32),
    )(degp, zoffs)


def _tc_msg(X, pa, pb, sa, sb):
    R = X.shape[0]

    def body(x, w1a, b1a, w2a, b2a, w1b, b1b, w2b, b2b, sa_r, sb_r, oa, ob):
        xv = x[...]
        ha = jnp.maximum(jnp.dot(xv, w1a[...], preferred_element_type=f32)
                         + b1a[...], 0.0)
        oa[...] = (jnp.dot(ha, w2a[...], preferred_element_type=f32)
                   + b2a[...]) * sa_r[...]
        hb = jnp.maximum(jnp.dot(xv, w1b[...], preferred_element_type=f32)
                         + b1b[...], 0.0)
        ob[...] = (jnp.dot(hb, w2b[...], preferred_element_type=f32)
                   + b2b[...]) * sb_r[...]

    row = lambda i: (i, 0)
    full = lambda i: (0, 0)
    wspec = pl.BlockSpec((DIM, DIM), full)
    bspec = pl.BlockSpec((1, DIM), full)
    dspec = pl.BlockSpec((BR, 1), row)
    return pl.pallas_call(
        body,
        grid=(R // BR,),
        in_specs=[pl.BlockSpec((BR, DIM), row),
                  wspec, bspec, wspec, bspec,
                  wspec, bspec, wspec, bspec,
                  dspec, dspec],
        out_specs=[pl.BlockSpec((BR, DIM), row)] * 2,
        out_shape=[jax.ShapeDtypeStruct((R, DIM), f32)] * 2,
    )(X, pa["W1"], pa["b1"].reshape(1, DIM), pa["W2"], pa["b2"].reshape(1, DIM),
      pb["W1"], pb["b1"].reshape(1, DIM), pb["W2"], pb["b2"].reshape(1, DIM),
      sa, sb)


def _tc_update(X, A, B, W, bvec, sa, sb):
    R = X.shape[0]

    def body(x, a, b, w0, w1, w2, bi, sa_r, sb_r, o):
        am = (a[0] + a[1]) * sa_r[...]
        bm = (b[0] + b[1]) * sb_r[...]
        o[...] = (jnp.dot(x[...], w0[...], preferred_element_type=f32)
                  + jnp.dot(am, w1[...], preferred_element_type=f32)
                  + jnp.dot(bm, w2[...], preferred_element_type=f32)
                  + bi[...])

    row = lambda i: (i, 0)
    full = lambda i: (0, 0)
    prow = lambda i: (0, i, 0)
    wspec = pl.BlockSpec((DIM, DIM), full)
    dspec = pl.BlockSpec((BR, 1), row)
    return pl.pallas_call(
        body,
        grid=(R // BR,),
        in_specs=[pl.BlockSpec((BR, DIM), row),
                  pl.BlockSpec((NC, BR, DIM), prow),
                  pl.BlockSpec((NC, BR, DIM), prow),
                  wspec, wspec, wspec,
                  pl.BlockSpec((1, DIM), full),
                  dspec, dspec],
        out_specs=pl.BlockSpec((BR, DIM), row),
        out_shape=jax.ShapeDtypeStruct((R, DIM), f32),
    )(X, A, B, W[0], W[1], W[2], bvec.reshape(1, DIM), sa, sb)


def kernel(v_size, c_size, v_edge_index, c_edge_index, p_edge_index,
           n_edge_index, v_emb, c_emb, params):
    V = v_emb.shape[0]
    Cn = c_emb.shape[0]
    E = v_edge_index.shape[0]
    EP = p_edge_index.shape[0]
    TRASH = max(V, Cn)
    R = -(-(TRASH + 1) // BR) * BR

    EPW = EP // NW
    NCH = -(-EPW // CHUNK)
    PADW = NCH * CHUNK - EPW

    ve_pad = jnp.concatenate(
        [v_edge_index.astype(i32), jnp.full((64,), TRASH, i32)])
    ce_pad = jnp.concatenate(
        [c_edge_index.astype(i32), jnp.full((64,), TRASH, i32)])
    p_idx = jnp.concatenate(
        [p_edge_index.astype(i32).reshape(NW, EPW),
         jnp.full((NW, PADW), E, i32)], axis=1).reshape(NW, NCH, CHUNK)
    n_idx = jnp.concatenate(
        [n_edge_index.astype(i32).reshape(NW, EPW),
         jnp.full((NW, PADW), E, i32)], axis=1).reshape(NW, NCH, CHUNK)

    zeros2d = jnp.zeros((32, DIM), f32)
    ones2d = jnp.ones((CHUNK, CHUNK), f32)

    vp, cp, vn, cn, degp = _sc_precompute(ve_pad, ce_pad, p_idx, n_idx,
                                          ones2d, zeros2d, R)

    v_zero = jnp.asarray(v_size - V, f32).reshape(1, 1)
    c_zero = jnp.asarray(c_size - Cn, f32).reshape(1, 1)
    zoffs = jnp.stack([v_zero, c_zero, v_zero, c_zero]).reshape(4, 1, 1)
    scales = _tc_scales(degp, zoffs, R)
    spv, spc, snv, snc = scales[0], scales[1], scales[2], scales[3]

    vW = params["vU_W"].reshape(3, DIM, DIM)
    cW = params["cU_W"].reshape(3, DIM, DIM)

    v_cur = jnp.concatenate([v_emb, jnp.zeros((R - V, DIM), f32)])
    c_cur = jnp.concatenate([c_emb, jnp.zeros((R - Cn, DIM), f32)])
    v_list = [v_cur]
    c_list = [c_cur]
    for _ in range(4):
        Mp, Mn = _tc_msg(v_cur, params["p_v2c"], params["n_v2c"], spv, snv)
        Mpc, Mnc = _tc_msg(c_cur, params["p_c2v"], params["n_c2v"], spc, snc)
        pA, nA, pC, nC = _sc_edge(Mp, Mn, Mpc, Mnc, vp, cp, vn, cn,
                                  zeros2d, R)
        c_cur = _tc_update(c_cur, pA, nA, cW, params["cU_b"], spc, snc)
        v_cur = _tc_update(v_cur, pC, nC, vW, params["vU_b"], spv, snv)
        c_list.append(c_cur)
        v_list.append(v_cur)

    v_out = jnp.stack(v_list)[:, :V, :]
    c_out = jnp.stack(c_list)[:, :Cn, :]
    return (v_out, c_out)


# speedup vs baseline: 4.1339x; 1.0782x over previous
import functools

import jax
import jax.numpy as jnp
from jax import lax
from jax.experimental import pallas as pl
from jax.experimental.pallas import tpu as pltpu
from jax.experimental.pallas import tpu_sc as plsc

f32 = jnp.float32
i32 = jnp.int32

NC, NS = 2, 16
NW = NC * NS
CHUNK = 128
DIM = 128
BR = 2560


def _mesh():
    return plsc.VectorSubcoreMesh(core_axis_name="c", subcore_axis_name="s")


def _sc_precompute(ve_pad, ce_pad, p_idx, n_idx, ones2d, zeros2d, R):
    NCH = p_idx.shape[1]
    rows = R // NS
    out_type = (
        jax.ShapeDtypeStruct((NW, NCH, CHUNK), i32),
        jax.ShapeDtypeStruct((NW, NCH, CHUNK), i32),
        jax.ShapeDtypeStruct((NW, NCH, CHUNK), i32),
        jax.ShapeDtypeStruct((NW, NCH, CHUNK), i32),
        jax.ShapeDtypeStruct((4, NC, R, CHUNK), f32),
    )
    scratch = [
        pltpu.VMEM((NCH, CHUNK), i32),
        pltpu.VMEM((NCH, CHUNK), i32),
        pltpu.VMEM((NCH, CHUNK), i32),
        pltpu.VMEM((CHUNK, CHUNK), f32),
        pltpu.VMEM((32, CHUNK), f32),
        pltpu.SemaphoreType.DMA,
        pltpu.VMEM_SHARED((R, CHUNK), f32),
    ]

    @functools.partial(pl.kernel, out_type=out_type, mesh=_mesh(),
                       scratch_types=scratch)
    def k(ve, ce, pidx, nidx, ones_h, zeros_h,
          vp_o, cp_o, vn_o, cn_o, deg_o,
          eidx_v, iv_v, ic_v, ones_v, z_v, sem, acc):
        cid = lax.axis_index("c")
        sid = lax.axis_index("s")
        w = sid * NC + cid
        base = sid * rows
        pltpu.sync_copy(ones_h, ones_v)
        pltpu.sync_copy(zeros_h, z_v)
        for kk in range(rows // 32):
            pltpu.async_copy(z_v, acc.at[pl.ds(base + kk * 32, 32), :], sem)
        for kk in range(rows // 32):
            pltpu.make_async_copy(z_v, acc.at[pl.ds(base + kk * 32, 32), :],
                                  sem).wait()
        plsc.subcore_barrier()

        for side, (idx_h, v_o, c_o) in enumerate(
                ((pidx, vp_o, cp_o), (nidx, vn_o, cn_o))):
            pltpu.sync_copy(idx_h.at[w], eidx_v)

            def gfire(j, carry):
                ej = eidx_v.at[j]
                pltpu.async_copy(ve.at[ej], iv_v.at[j], sem)
                pltpu.async_copy(ce.at[ej], ic_v.at[j], sem)
                return carry

            def gdrain(j, carry):
                ej = eidx_v.at[j]
                pltpu.make_async_copy(ve.at[ej], iv_v.at[j], sem).wait()
                pltpu.make_async_copy(ce.at[ej], ic_v.at[j], sem).wait()
                return carry

            lax.fori_loop(0, NCH, gfire, 0)
            lax.fori_loop(0, NCH, gdrain, 0)
            pltpu.sync_copy(iv_v, v_o.at[w])
            pltpu.sync_copy(ic_v, c_o.at[w])

            for half, idxref in ((0, iv_v), (1, ic_v)):
                a = 2 * side + half

                def hfire(j, carry, idxref=idxref):
                    pltpu.async_copy(ones_v, acc.at[idxref.at[j]], sem,
                                     add=True)
                    return carry

                def hdrain(j, carry, idxref=idxref):
                    pltpu.make_async_copy(ones_v, acc.at[idxref.at[j]],
                                          sem).wait()
                    return carry

                lax.fori_loop(0, NCH, hfire, 0)
                lax.fori_loop(0, NCH, hdrain, 0)
                plsc.subcore_barrier()
                for kk in range(rows // CHUNK):
                    sl = pl.ds(base + kk * CHUNK, CHUNK)
                    pltpu.async_copy(acc.at[sl, :], deg_o.at[a, cid, sl, :],
                                     sem)
                for kk in range(rows // CHUNK):
                    sl = pl.ds(base + kk * CHUNK, CHUNK)
                    pltpu.make_async_copy(acc.at[sl, :],
                                          deg_o.at[a, cid, sl, :], sem).wait()
                for kk in range(rows // 32):
                    pltpu.async_copy(z_v, acc.at[pl.ds(base + kk * 32, 32),
                                                 :], sem)
                for kk in range(rows // 32):
                    pltpu.make_async_copy(z_v, acc.at[pl.ds(base + kk * 32,
                                                            32), :],
                                          sem).wait()
                plsc.subcore_barrier()

    return k(ve_pad, ce_pad, p_idx, n_idx, ones2d, zeros2d)


def _sc_edge(Mp, Mn, Mpc, Mnc, vp, cp, vn, cn, zeros2d, R):
    NCH = vp.shape[1]
    rows = R // NS
    out_type = tuple(jax.ShapeDtypeStruct((NC, R, DIM), f32) for _ in range(4))
    scratch = [
        pltpu.VMEM((NCH, CHUNK), i32),
        pltpu.VMEM((NCH, CHUNK), i32),
        pltpu.VMEM((2, CHUNK, DIM), f32),
        pltpu.VMEM((32, DIM), f32),
        pltpu.SemaphoreType.DMA,
        pltpu.SemaphoreType.DMA,
        pltpu.VMEM_SHARED((R, DIM), f32),
    ]

    @functools.partial(pl.kernel, out_type=out_type, mesh=_mesh(),
                       scratch_types=scratch)
    def k(mp, mn, mpc, mnc, vp_h, cp_h, vn_h, cn_h, z_h,
          pA, nA, pC, nC,
          isrc, idst, rv, z_v, gsem, ssem, acc):
        cid = lax.axis_index("c")
        sid = lax.axis_index("s")
        w = sid * NC + cid
        base = sid * rows
        pltpu.sync_copy(z_h, z_v)
        for kk in range(rows // 32):
            pltpu.async_copy(z_v, acc.at[pl.ds(base + kk * 32, 32), :], ssem)
        for kk in range(rows // 32):
            pltpu.make_async_copy(z_v, acc.at[pl.ds(base + kk * 32, 32), :],
                                  ssem).wait()
        plsc.subcore_barrier()
        for tab, s_h, d_h, out in ((mp, vp_h, cp_h, pA), (mpc, cp_h, vp_h, pC),
                                   (mn, vn_h, cn_h, nA), (mnc, cn_h, vn_h, nC)):
            pltpu.sync_copy(s_h.at[w], isrc)
            pltpu.sync_copy(d_h.at[w], idst)
            pltpu.async_copy(tab.at[isrc.at[0]], rv.at[0], gsem)

            def body(j, carry, tab=tab):
                ph = lax.bitwise_and(j, 1)

                @pl.when(j > 0)
                def _():
                    pltpu.make_async_copy(rv.at[1 - ph],
                                          acc.at[idst.at[j - 1]], ssem).wait()

                @pl.when(j + 1 < NCH)
                def _():
                    pltpu.async_copy(tab.at[isrc.at[j + 1]], rv.at[1 - ph],
                                     gsem)

                pltpu.make_async_copy(tab.at[isrc.at[j]], rv.at[ph],
                                      gsem).wait()
                pltpu.async_copy(rv.at[ph], acc.at[idst.at[j]], ssem,
                                 add=True)
                return carry

            lax.fori_loop(0, NCH, body, 0)
            lastph = lax.bitwise_and(NCH - 1, 1)
            pltpu.make_async_copy(rv.at[lastph], acc.at[idst.at[NCH - 1]],
                                  ssem).wait()
            plsc.subcore_barrier()
            for kk in range(rows // CHUNK):
                sl = pl.ds(base + kk * CHUNK, CHUNK)
                pltpu.async_copy(acc.at[sl, :], out.at[cid].at[sl, :], gsem)
            for kk in range(rows // CHUNK):
                sl = pl.ds(base + kk * CHUNK, CHUNK)
                pltpu.make_async_copy(acc.at[sl, :], out.at[cid].at[sl, :],
                                      gsem).wait()
            for kk in range(rows // 32):
                pltpu.async_copy(z_v, acc.at[pl.ds(base + kk * 32, 32), :],
                                 ssem)
            for kk in range(rows // 32):
                pltpu.make_async_copy(z_v, acc.at[pl.ds(base + kk * 32, 32),
                                                  :], ssem).wait()
            plsc.subcore_barrier()

    return k(Mp, Mn, Mpc, Mnc, vp, cp, vn, cn, zeros2d)


def _tc_scales(degp, zoffs, R):

    def body(d, z, o):
        dv = d[...]
        cnt = dv[:, 0, :, 0:1] + dv[:, 1, :, 0:1] + z[...]
        o[...] = lax.rsqrt(jnp.maximum(cnt, 1.0))

    return pl.pallas_call(
        body,
        grid=(R // BR,),
        in_specs=[pl.BlockSpec((4, NC, BR, CHUNK), lambda i: (0, 0, i, 0)),
                  pl.BlockSpec((4, 1, 1), lambda i: (0, 0, 0))],
        out_specs=pl.BlockSpec((4, BR, 1), lambda i: (0, i, 0)),
        out_shape=jax.ShapeDtypeStruct((4, R, 1), f32),
    )(degp, zoffs)


def _tc_msg(X, pa, pb, sa, sb):
    R = X.shape[0]

    def body(x, w1a, b1a, w2a, b2a, w1b, b1b, w2b, b2b, sa_r, sb_r, oa, ob):
        xv = x[...]
        ha = jnp.maximum(jnp.dot(xv, w1a[...], preferred_element_type=f32)
                         + b1a[...], 0.0)
        oa[...] = (jnp.dot(ha, w2a[...], preferred_element_type=f32)
                   + b2a[...]) * sa_r[...]
        hb = jnp.maximum(jnp.dot(xv, w1b[...], preferred_element_type=f32)
                         + b1b[...], 0.0)
        ob[...] = (jnp.dot(hb, w2b[...], preferred_element_type=f32)
                   + b2b[...]) * sb_r[...]

    row = lambda i: (i, 0)
    full = lambda i: (0, 0)
    wspec = pl.BlockSpec((DIM, DIM), full)
    bspec = pl.BlockSpec((1, DIM), full)
    dspec = pl.BlockSpec((BR, 1), row)
    return pl.pallas_call(
        body,
        grid=(R // BR,),
        in_specs=[pl.BlockSpec((BR, DIM), row),
                  wspec, bspec, wspec, bspec,
                  wspec, bspec, wspec, bspec,
                  dspec, dspec],
        out_specs=[pl.BlockSpec((BR, DIM), row)] * 2,
        out_shape=[jax.ShapeDtypeStruct((R, DIM), f32)] * 2,
    )(X, pa["W1"], pa["b1"].reshape(1, DIM), pa["W2"], pa["b2"].reshape(1, DIM),
      pb["W1"], pb["b1"].reshape(1, DIM), pb["W2"], pb["b2"].reshape(1, DIM),
      sa, sb)


def _tc_update(X, A, B, W, bvec, sa, sb):
    R = X.shape[0]

    def body(x, a, b, w0, w1, w2, bi, sa_r, sb_r, o):
        am = (a[0] + a[1]) * sa_r[...]
        bm = (b[0] + b[1]) * sb_r[...]
        o[...] = (jnp.dot(x[...], w0[...], preferred_element_type=f32)
                  + jnp.dot(am, w1[...], preferred_element_type=f32)
                  + jnp.dot(bm, w2[...], preferred_element_type=f32)
                  + bi[...])

    row = lambda i: (i, 0)
    full = lambda i: (0, 0)
    prow = lambda i: (0, i, 0)
    wspec = pl.BlockSpec((DIM, DIM), full)
    dspec = pl.BlockSpec((BR, 1), row)
    return pl.pallas_call(
        body,
        grid=(R // BR,),
        in_specs=[pl.BlockSpec((BR, DIM), row),
                  pl.BlockSpec((NC, BR, DIM), prow),
                  pl.BlockSpec((NC, BR, DIM), prow),
                  wspec, wspec, wspec,
                  pl.BlockSpec((1, DIM), full),
                  dspec, dspec],
        out_specs=pl.BlockSpec((BR, DIM), row),
        out_shape=jax.ShapeDtypeStruct((R, DIM), f32),
    )(X, A, B, W[0], W[1], W[2], bvec.reshape(1, DIM), sa, sb)


def kernel(v_size, c_size, v_edge_index, c_edge_index, p_edge_index,
           n_edge_index, v_emb, c_emb, params):
    V = v_emb.shape[0]
    Cn = c_emb.shape[0]
    E = v_edge_index.shape[0]
    EP = p_edge_index.shape[0]
    TRASH = max(V, Cn)
    R = -(-(TRASH + 1) // BR) * BR

    EPW = EP // NW
    NCH = -(-EPW // CHUNK)
    PADW = NCH * CHUNK - EPW

    ve_pad = jnp.concatenate(
        [v_edge_index.astype(i32), jnp.full((64,), TRASH, i32)])
    ce_pad = jnp.concatenate(
        [c_edge_index.astype(i32), jnp.full((64,), TRASH, i32)])
    p_idx = jnp.concatenate(
        [p_edge_index.astype(i32).reshape(NW, EPW),
         jnp.full((NW, PADW), E, i32)], axis=1).reshape(NW, NCH, CHUNK)
    n_idx = jnp.concatenate(
        [n_edge_index.astype(i32).reshape(NW, EPW),
         jnp.full((NW, PADW), E, i32)], axis=1).reshape(NW, NCH, CHUNK)

    zeros2d = jnp.zeros((32, DIM), f32)
    ones2d = jnp.ones((CHUNK, CHUNK), f32)

    vp, cp, vn, cn, degp = _sc_precompute(ve_pad, ce_pad, p_idx, n_idx,
                                          ones2d, zeros2d, R)

    v_zero = jnp.asarray(v_size - V, f32).reshape(1, 1)
    c_zero = jnp.asarray(c_size - Cn, f32).reshape(1, 1)
    zoffs = jnp.stack([v_zero, c_zero, v_zero, c_zero]).reshape(4, 1, 1)
    scales = _tc_scales(degp, zoffs, R)
    spv, spc, snv, snc = scales[0], scales[1], scales[2], scales[3]

    vW = params["vU_W"].reshape(3, DIM, DIM)
    cW = params["cU_W"].reshape(3, DIM, DIM)

    v_cur = jnp.concatenate([v_emb, jnp.zeros((R - V, DIM), f32)])
    c_cur = jnp.concatenate([c_emb, jnp.zeros((R - Cn, DIM), f32)])
    v_list = [v_cur]
    c_list = [c_cur]
    for _ in range(4):
        Mp, Mn = _tc_msg(v_cur, params["p_v2c"], params["n_v2c"], spv, snv)
        Mpc, Mnc = _tc_msg(c_cur, params["p_c2v"], params["n_c2v"], spc, snc)
        pA, nA, pC, nC = _sc_edge(Mp, Mn, Mpc, Mnc, vp, cp, vn, cn,
                                  zeros2d, R)
        c_cur = _tc_update(c_cur, pA, nA, cW, params["cU_b"], spc, snc)
        v_cur = _tc_update(v_cur, pC, nC, vW, params["vU_b"], spv, snv)
        c_list.append(c_cur)
        v_list.append(v_cur)

    v_out = jnp.stack(v_list)[:, :V, :]
    c_out = jnp.stack(c_list)[:, :Cn, :]
    return (v_out, c_out)


# speedup vs baseline: 4.1687x; 1.0084x over previous
import functools

import jax
import jax.numpy as jnp
from jax import lax
from jax.experimental import pallas as pl
from jax.experimental.pallas import tpu as pltpu
from jax.experimental.pallas import tpu_sc as plsc

f32 = jnp.float32
i32 = jnp.int32

NC, NS = 2, 16
NW = NC * NS
CHUNK = 128
DIM = 128
BR = 2560


def _mesh():
    return plsc.VectorSubcoreMesh(core_axis_name="c", subcore_axis_name="s")


def _sc_precompute(ve_pad, ce_pad, p_idx, n_idx, ones2d, zeros2d, R):
    NCH = p_idx.shape[1]
    rows = R // NS
    out_type = (
        jax.ShapeDtypeStruct((NW, NCH, CHUNK), i32),
        jax.ShapeDtypeStruct((NW, NCH, CHUNK), i32),
        jax.ShapeDtypeStruct((NW, NCH, CHUNK), i32),
        jax.ShapeDtypeStruct((NW, NCH, CHUNK), i32),
        jax.ShapeDtypeStruct((4, NC, R, CHUNK), f32),
    )
    scratch = [
        pltpu.VMEM((NCH, CHUNK), i32),
        pltpu.VMEM((NCH, CHUNK), i32),
        pltpu.VMEM((NCH, CHUNK), i32),
        pltpu.VMEM((CHUNK, CHUNK), f32),
        pltpu.VMEM((32, CHUNK), f32),
        pltpu.SemaphoreType.DMA,
        pltpu.VMEM_SHARED((R, CHUNK), f32),
    ]

    @functools.partial(pl.kernel, out_type=out_type, mesh=_mesh(),
                       scratch_types=scratch)
    def k(ve, ce, pidx, nidx, ones_h, zeros_h,
          vp_o, cp_o, vn_o, cn_o, deg_o,
          eidx_v, iv_v, ic_v, ones_v, z_v, sem, acc):
        cid = lax.axis_index("c")
        sid = lax.axis_index("s")
        w = sid * NC + cid
        base = sid * rows
        pltpu.sync_copy(ones_h, ones_v)
        pltpu.sync_copy(zeros_h, z_v)
        for kk in range(rows // 32):
            pltpu.async_copy(z_v, acc.at[pl.ds(base + kk * 32, 32), :], sem)
        for kk in range(rows // 32):
            pltpu.make_async_copy(z_v, acc.at[pl.ds(base + kk * 32, 32), :],
                                  sem).wait()
        plsc.subcore_barrier()

        for side, (idx_h, v_o, c_o) in enumerate(
                ((pidx, vp_o, cp_o), (nidx, vn_o, cn_o))):
            pltpu.sync_copy(idx_h.at[w], eidx_v)

            def gfire(j, carry):
                ej = eidx_v.at[j]
                pltpu.async_copy(ve.at[ej], iv_v.at[j], sem)
                pltpu.async_copy(ce.at[ej], ic_v.at[j], sem)
                return carry

            def gdrain(j, carry):
                ej = eidx_v.at[j]
                pltpu.make_async_copy(ve.at[ej], iv_v.at[j], sem).wait()
                pltpu.make_async_copy(ce.at[ej], ic_v.at[j], sem).wait()
                return carry

            lax.fori_loop(0, NCH, gfire, 0)
            lax.fori_loop(0, NCH, gdrain, 0)
            pltpu.sync_copy(iv_v, v_o.at[w])
            pltpu.sync_copy(ic_v, c_o.at[w])

            for half, idxref in ((0, iv_v), (1, ic_v)):
                a = 2 * side + half

                def hfire(j, carry, idxref=idxref):
                    pltpu.async_copy(ones_v, acc.at[idxref.at[j]], sem,
                                     add=True)
                    return carry

                def hdrain(j, carry, idxref=idxref):
                    pltpu.make_async_copy(ones_v, acc.at[idxref.at[j]],
                                          sem).wait()
                    return carry

                lax.fori_loop(0, NCH, hfire, 0)
                lax.fori_loop(0, NCH, hdrain, 0)
                plsc.subcore_barrier()
                for kk in range(rows // CHUNK):
                    sl = pl.ds(base + kk * CHUNK, CHUNK)
                    pltpu.async_copy(acc.at[sl, :], deg_o.at[a, cid, sl, :],
                                     sem)
                for kk in range(rows // CHUNK):
                    sl = pl.ds(base + kk * CHUNK, CHUNK)
                    pltpu.make_async_copy(acc.at[sl, :],
                                          deg_o.at[a, cid, sl, :], sem).wait()
                for kk in range(rows // 32):
                    pltpu.async_copy(z_v, acc.at[pl.ds(base + kk * 32, 32),
                                                 :], sem)
                for kk in range(rows // 32):
                    pltpu.make_async_copy(z_v, acc.at[pl.ds(base + kk * 32,
                                                            32), :],
                                          sem).wait()
                plsc.subcore_barrier()

    return k(ve_pad, ce_pad, p_idx, n_idx, ones2d, zeros2d)


def _sc_edge(Mp, Mn, Mpc, Mnc, vp, cp, vn, cn, zeros2d, R):
    NCH = vp.shape[1]
    rows = R // NS
    out_type = tuple(jax.ShapeDtypeStruct((NC, R, DIM), f32) for _ in range(4))
    scratch = [
        pltpu.VMEM((NCH, CHUNK), i32),
        pltpu.VMEM((NCH, CHUNK), i32),
        pltpu.VMEM((2, CHUNK, DIM), f32),
        pltpu.VMEM((32, DIM), f32),
        pltpu.SemaphoreType.DMA,
        pltpu.SemaphoreType.DMA,
        pltpu.VMEM_SHARED((R, DIM), f32),
    ]

    @functools.partial(pl.kernel, out_type=out_type, mesh=_mesh(),
                       scratch_types=scratch)
    def k(mp, mn, mpc, mnc, vp_h, cp_h, vn_h, cn_h, z_h,
          pA, nA, pC, nC,
          isrc, idst, rv, z_v, gsem, ssem, acc):
        cid = lax.axis_index("c")
        sid = lax.axis_index("s")
        w = sid * NC + cid
        base = sid * rows
        pltpu.sync_copy(z_h, z_v)
        first = True
        for tab, s_h, d_h, out in ((mp, vp_h, cp_h, pA), (mpc, cp_h, vp_h, pC),
                                   (mn, vn_h, cn_h, nA), (mnc, cn_h, vn_h, nC)):
            for kk in range(rows // 32):
                pltpu.async_copy(z_v, acc.at[pl.ds(base + kk * 32, 32), :],
                                 ssem)
            pltpu.sync_copy(s_h.at[w], isrc)
            pltpu.sync_copy(d_h.at[w], idst)
            for kk in range(rows // 32):
                pltpu.make_async_copy(z_v, acc.at[pl.ds(base + kk * 32, 32),
                                                  :], ssem).wait()
            plsc.subcore_barrier()
            pltpu.async_copy(tab.at[isrc.at[0]], rv.at[0], gsem)

            def body(j, carry, tab=tab):
                ph = lax.bitwise_and(j, 1)

                @pl.when(j > 0)
                def _():
                    pltpu.make_async_copy(rv.at[1 - ph],
                                          acc.at[idst.at[j - 1]], ssem).wait()

                @pl.when(j + 1 < NCH)
                def _():
                    pltpu.async_copy(tab.at[isrc.at[j + 1]], rv.at[1 - ph],
                                     gsem)

                pltpu.make_async_copy(tab.at[isrc.at[j]], rv.at[ph],
                                      gsem).wait()
                pltpu.async_copy(rv.at[ph], acc.at[idst.at[j]], ssem,
                                 add=True)
                return carry

            lax.fori_loop(0, NCH, body, 0)
            lastph = lax.bitwise_and(NCH - 1, 1)
            pltpu.make_async_copy(rv.at[lastph], acc.at[idst.at[NCH - 1]],
                                  ssem).wait()
            plsc.subcore_barrier()
            for kk in range(rows // CHUNK):
                sl = pl.ds(base + kk * CHUNK, CHUNK)
                pltpu.async_copy(acc.at[sl, :], out.at[cid].at[sl, :], gsem)
            for kk in range(rows // CHUNK):
                sl = pl.ds(base + kk * CHUNK, CHUNK)
                pltpu.make_async_copy(acc.at[sl, :], out.at[cid].at[sl, :],
                                      gsem).wait()
            plsc.subcore_barrier()

    return k(Mp, Mn, Mpc, Mnc, vp, cp, vn, cn, zeros2d)


def _tc_scales(degp, zoffs, R):

    def body(d, z, o):
        dv = d[...]
        cnt = dv[:, 0, :, 0:1] + dv[:, 1, :, 0:1] + z[...]
        o[...] = lax.rsqrt(jnp.maximum(cnt, 1.0))

    return pl.pallas_call(
        body,
        grid=(R // BR,),
        in_specs=[pl.BlockSpec((4, NC, BR, CHUNK), lambda i: (0, 0, i, 0)),
                  pl.BlockSpec((4, 1, 1), lambda i: (0, 0, 0))],
        out_specs=pl.BlockSpec((4, BR, 1), lambda i: (0, i, 0)),
        out_shape=jax.ShapeDtypeStruct((4, R, 1), f32),
    )(degp, zoffs)


def _tc_msg(X, pa, pb, sa, sb):
    R = X.shape[0]

    def body(x, w1a, b1a, w2a, b2a, w1b, b1b, w2b, b2b, sa_r, sb_r, oa, ob):
        xv = x[...]
        ha = jnp.maximum(jnp.dot(xv, w1a[...], preferred_element_type=f32)
                         + b1a[...], 0.0)
        oa[...] = (jnp.dot(ha, w2a[...], preferred_element_type=f32)
                   + b2a[...]) * sa_r[...]
        hb = jnp.maximum(jnp.dot(xv, w1b[...], preferred_element_type=f32)
                         + b1b[...], 0.0)
        ob[...] = (jnp.dot(hb, w2b[...], preferred_element_type=f32)
                   + b2b[...]) * sb_r[...]

    row = lambda i: (i, 0)
    full = lambda i: (0, 0)
    wspec = pl.BlockSpec((DIM, DIM), full)
    bspec = pl.BlockSpec((1, DIM), full)
    dspec = pl.BlockSpec((BR, 1), row)
    return pl.pallas_call(
        body,
        grid=(R // BR,),
        in_specs=[pl.BlockSpec((BR, DIM), row),
                  wspec, bspec, wspec, bspec,
                  wspec, bspec, wspec, bspec,
                  dspec, dspec],
        out_specs=[pl.BlockSpec((BR, DIM), row)] * 2,
        out_shape=[jax.ShapeDtypeStruct((R, DIM), f32)] * 2,
    )(X, pa["W1"], pa["b1"].reshape(1, DIM), pa["W2"], pa["b2"].reshape(1, DIM),
      pb["W1"], pb["b1"].reshape(1, DIM), pb["W2"], pb["b2"].reshape(1, DIM),
      sa, sb)


def _tc_update(X, A, B, W, bvec, sa, sb):
    R = X.shape[0]

    def body(x, a, b, w0, w1, w2, bi, sa_r, sb_r, o):
        am = (a[0] + a[1]) * sa_r[...]
        bm = (b[0] + b[1]) * sb_r[...]
        o[...] = (jnp.dot(x[...], w0[...], preferred_element_type=f32)
                  + jnp.dot(am, w1[...], preferred_element_type=f32)
                  + jnp.dot(bm, w2[...], preferred_element_type=f32)
                  + bi[...])

    row = lambda i: (i, 0)
    full = lambda i: (0, 0)
    prow = lambda i: (0, i, 0)
    wspec = pl.BlockSpec((DIM, DIM), full)
    dspec = pl.BlockSpec((BR, 1), row)
    return pl.pallas_call(
        body,
        grid=(R // BR,),
        in_specs=[pl.BlockSpec((BR, DIM), row),
                  pl.BlockSpec((NC, BR, DIM), prow),
                  pl.BlockSpec((NC, BR, DIM), prow),
                  wspec, wspec, wspec,
                  pl.BlockSpec((1, DIM), full),
                  dspec, dspec],
        out_specs=pl.BlockSpec((BR, DIM), row),
        out_shape=jax.ShapeDtypeStruct((R, DIM), f32),
    )(X, A, B, W[0], W[1], W[2], bvec.reshape(1, DIM), sa, sb)


def kernel(v_size, c_size, v_edge_index, c_edge_index, p_edge_index,
           n_edge_index, v_emb, c_emb, params):
    V = v_emb.shape[0]
    Cn = c_emb.shape[0]
    E = v_edge_index.shape[0]
    EP = p_edge_index.shape[0]
    TRASH = max(V, Cn)
    R = -(-(TRASH + 1) // BR) * BR

    EPW = EP // NW
    NCH = -(-EPW // CHUNK)
    PADW = NCH * CHUNK - EPW

    ve_pad = jnp.concatenate(
        [v_edge_index.astype(i32), jnp.full((64,), TRASH, i32)])
    ce_pad = jnp.concatenate(
        [c_edge_index.astype(i32), jnp.full((64,), TRASH, i32)])
    p_idx = jnp.concatenate(
        [p_edge_index.astype(i32).reshape(NW, EPW),
         jnp.full((NW, PADW), E, i32)], axis=1).reshape(NW, NCH, CHUNK)
    n_idx = jnp.concatenate(
        [n_edge_index.astype(i32).reshape(NW, EPW),
         jnp.full((NW, PADW), E, i32)], axis=1).reshape(NW, NCH, CHUNK)

    zeros2d = jnp.zeros((32, DIM), f32)
    ones2d = jnp.ones((CHUNK, CHUNK), f32)

    vp, cp, vn, cn, degp = _sc_precompute(ve_pad, ce_pad, p_idx, n_idx,
                                          ones2d, zeros2d, R)

    v_zero = jnp.asarray(v_size - V, f32).reshape(1, 1)
    c_zero = jnp.asarray(c_size - Cn, f32).reshape(1, 1)
    zoffs = jnp.stack([v_zero, c_zero, v_zero, c_zero]).reshape(4, 1, 1)
    scales = _tc_scales(degp, zoffs, R)
    spv, spc, snv, snc = scales[0], scales[1], scales[2], scales[3]

    vW = params["vU_W"].reshape(3, DIM, DIM)
    cW = params["cU_W"].reshape(3, DIM, DIM)

    v_cur = jnp.concatenate([v_emb, jnp.zeros((R - V, DIM), f32)])
    c_cur = jnp.concatenate([c_emb, jnp.zeros((R - Cn, DIM), f32)])
    v_list = [v_cur]
    c_list = [c_cur]
    for _ in range(4):
        Mp, Mn = _tc_msg(v_cur, params["p_v2c"], params["n_v2c"], spv, snv)
        Mpc, Mnc = _tc_msg(c_cur, params["p_c2v"], params["n_c2v"], spc, snc)
        pA, nA, pC, nC = _sc_edge(Mp, Mn, Mpc, Mnc, vp, cp, vn, cn,
                                  zeros2d, R)
        c_cur = _tc_update(c_cur, pA, nA, cW, params["cU_b"], spc, snc)
        v_cur = _tc_update(v_cur, pC, nC, vW, params["vU_b"], spv, snv)
        c_list.append(c_cur)
        v_list.append(v_cur)

    v_out = jnp.stack(v_list)[:, :V, :]
    c_out = jnp.stack(c_list)[:, :Cn, :]
    return (v_out, c_out)
